# baseline (speedup 1.0000x reference)
import functools

import jax
import jax.numpy as jnp
from jax import lax
from jax.experimental import pallas as pl
from jax.experimental.pallas import tpu as pltpu
from jax.experimental.pallas import tpu_sc as plsc

N = 10000
E = 320000
D = 128
G = 256

NC, NS = 2, 16
NW = NC * NS
CH = 128
E_PAD = 327680
NROWS2D = E_PAD // CH
CH_PER_TILE = NROWS2D // NW
NB = 2
SUP = 8
NSUPER = CH_PER_TILE // SUP
N_PAD = 10240
ROWS_PER_TILE = N_PAD // NS

_SC_MESH = plsc.VectorSubcoreMesh(
    core_axis_name="c", subcore_axis_name="s", num_cores=NC, num_subcores=NS)


def _sc_segsum_body(h_hbm, src_hbm, dst_hbm, z_hbm, out_hbm,
                    sidx, didx, rows, acc, gsems, ssems):
    cid = lax.axis_index("c")
    sid = lax.axis_index("s")
    wid = sid * NC + cid

    stripe = pl.multiple_of(sid * ROWS_PER_TILE, ROWS_PER_TILE)
    pltpu.sync_copy(z_hbm, acc.at[pl.ds(stripe, ROWS_PER_TILE)])

    tbase = pl.multiple_of(wid * CH_PER_TILE, CH_PER_TILE)
    plsc.subcore_barrier()

    def super_body(s, carry):
        rbase = pl.multiple_of(tbase + s * SUP, SUP)
        pltpu.sync_copy(src_hbm.at[pl.ds(rbase, SUP)], sidx)
        pltpu.sync_copy(dst_hbm.at[pl.ds(rbase, SUP)], didx)
        gd = [None] * NB
        sd = [None] * NB
        for j in range(SUP):
            b = j % NB
            if sd[b] is not None:
                sd[b].wait()
            gd[b] = pltpu.async_copy(h_hbm.at[sidx.at[j]], rows.at[b], gsems[b])
            if j >= 1:
                pb = (j - 1) % NB
                gd[pb].wait()
                sd[pb] = pltpu.async_copy(
                    rows.at[pb], acc.at[didx.at[j - 1]], ssems[pb], add=True)
        lb = (SUP - 1) % NB
        gd[lb].wait()
        sd[lb] = pltpu.async_copy(
            rows.at[lb], acc.at[didx.at[SUP - 1]], ssems[lb], add=True)
        for b in range(NB):
            sd[b].wait()
        return carry

    lax.fori_loop(0, NSUPER, super_body, 0)
    plsc.subcore_barrier()

    pltpu.sync_copy(acc.at[pl.ds(stripe, ROWS_PER_TILE)],
                    out_hbm.at[cid, pl.ds(stripe, ROWS_PER_TILE)])


_sc_segsum = pl.kernel(
    _sc_segsum_body,
    out_type=jax.ShapeDtypeStruct((NC, N_PAD, D), jnp.float32),
    mesh=_SC_MESH,
    scratch_types=[
        pltpu.VMEM((SUP, CH), jnp.int32),
        pltpu.VMEM((SUP, CH), jnp.int32),
        pltpu.VMEM((NB, CH, D), jnp.float32),
        pltpu.VMEM_SHARED((N_PAD, D), jnp.float32),
        [pltpu.SemaphoreType.DMA] * NB,
        [pltpu.SemaphoreType.DMA] * NB,
    ],
    name="sc_segsum",
)


def _gin_mlp_body(hp, agg, eps, w1, b1, g1, be1, w2, b2, g2, be2, out):
    z = hp[...] * (1.0 + eps[0, 0]) + agg[0, :N, :] + agg[1, :N, :]
    for (w, b, g, be) in ((w1, b1, g1, be1), (w2, b2, g2, be2)):
        y = jnp.dot(z, w[...], preferred_element_type=jnp.float32) + b[...]
        m = jnp.mean(y, axis=0, keepdims=True)
        v = jnp.mean((y - m) ** 2, axis=0, keepdims=True)
        y = (y - m) * lax.rsqrt(v + 1e-5) * g[...] + be[...]
        z = jnp.where(y > 0, y, 0.01 * y)
    out[...] = jnp.where(z > 0, z, 0.01 * z)


_gin_mlp = pl.pallas_call(
    _gin_mlp_body,
    out_shape=jax.ShapeDtypeStruct((N, D), jnp.float32),
)


def _cls_body(h1, h2, h3, h4, bi, w640, b1r, wa, ba, wb, bbr, fw, fb, out):
    onehot = (lax.broadcasted_iota(jnp.int32, (N, G), 1) == bi[...])
    p = onehot.astype(jnp.float32)
    h4v = h4[...]
    pool = lax.dot_general(p, h4v, (((0,), (0,)), ((), ())),
                           preferred_element_type=jnp.float32)
    pooled = jnp.dot(p, pool, preferred_element_type=jnp.float32)
    w = w640[...]
    h = (jnp.dot(h1[...], w[0:128], preferred_element_type=jnp.float32)
         + jnp.dot(h2[...], w[128:256], preferred_element_type=jnp.float32)
         + jnp.dot(h3[...], w[256:384], preferred_element_type=jnp.float32)
         + jnp.dot(h4v, w[384:512], preferred_element_type=jnp.float32)
         + jnp.dot(pooled, w[512:640], preferred_element_type=jnp.float32)
         + b1r[...])
    for (wh, bh) in ((wa, ba), (wb, bbr)):
        h = jnp.dot(h, wh[...], preferred_element_type=jnp.float32) + bh[...]
        h = jnp.where(h > 0, h, 0.01 * h)
    o = jnp.sum(h * fw[...], axis=1, keepdims=True) + fb[0, 0]
    out[...] = 1.0 / (1.0 + jnp.exp(-o))


_classifier = pl.pallas_call(
    _cls_body,
    out_shape=jax.ShapeDtypeStruct((N, 1), jnp.float32),
)


def kernel(x, edge_index, batch, params):
    src = jnp.concatenate(
        [edge_index[0].astype(jnp.int32),
         jnp.zeros((E_PAD - E,), jnp.int32)]).reshape(NROWS2D, CH)
    dst = jnp.concatenate(
        [edge_index[1].astype(jnp.int32),
         N + jnp.arange(E_PAD - E, dtype=jnp.int32) % (N_PAD - N)]
    ).reshape(NROWS2D, CH)
    zeros = jnp.zeros((ROWS_PER_TILE, D), jnp.float32)
    bi = batch.astype(jnp.int32).reshape(N, 1)

    def row(v):
        return v.reshape(1, -1)

    h = x
    hs = []
    for i in range(4):
        layers = params['conv_mlps'][i]
        agg2 = _sc_segsum(h, src, dst, zeros)
        l1, l2 = layers[0], layers[1]
        h = _gin_mlp(h, agg2, params['eps'][i].reshape(1, 1),
                     l1['W'], row(l1['b']), row(l1['gamma']), row(l1['beta']),
                     l2['W'], row(l2['b']), row(l2['gamma']), row(l2['beta']))
        hs.append(h)

    out = _classifier(hs[0], hs[1], hs[2], hs[3], bi,
                      params['cls1_W'], row(params['cls1_b']),
                      params['cls_Ws'][0], row(params['cls_bs'][0]),
                      params['cls_Ws'][1], row(params['cls_bs'][1]),
                      row(params['fin_W'][:, 0]), params['fin_b'].reshape(1, 1))
    return out


# speedup vs baseline: 3.3283x; 3.3283x over previous
import functools

import jax
import jax.numpy as jnp
from jax import lax
from jax.experimental import pallas as pl
from jax.experimental.pallas import tpu as pltpu
from jax.experimental.pallas import tpu_sc as plsc

N = 10000
E = 320000
D = 128
G = 256

NC, NS = 2, 16
NW = NC * NS
CH = 128
E_PAD = 327680
NROWS2D = E_PAD // CH
CH_PER_TILE = NROWS2D // NW
NB = 2
SUP = 8
NSUPER = CH_PER_TILE // SUP
N_PAD = 10240
ROWS_PER_TILE = N_PAD // NS

_SC_MESH = plsc.VectorSubcoreMesh(
    core_axis_name="c", subcore_axis_name="s", num_cores=NC, num_subcores=NS)


def _sc_segsum_body(h_hbm, src_hbm, dst_hbm, z_hbm, out_hbm,
                    sidx, didx, rows, acc, gsems, ssems):
    cid = lax.axis_index("c")
    sid = lax.axis_index("s")
    wid = sid * NC + cid

    stripe = pl.multiple_of(sid * ROWS_PER_TILE, ROWS_PER_TILE)
    pltpu.sync_copy(z_hbm, acc.at[pl.ds(stripe, ROWS_PER_TILE)])

    tbase = pl.multiple_of(wid * CH_PER_TILE, CH_PER_TILE)
    plsc.subcore_barrier()

    def super_body(s, carry):
        rbase = pl.multiple_of(tbase + s * SUP, SUP)
        pltpu.sync_copy(src_hbm.at[pl.ds(rbase, SUP)], sidx)
        pltpu.sync_copy(dst_hbm.at[pl.ds(rbase, SUP)], didx)
        gd = [None] * NB
        sd = [None] * NB
        for j in range(SUP):
            b = j % NB
            if sd[b] is not None:
                sd[b].wait()
            gd[b] = pltpu.async_copy(h_hbm.at[sidx.at[j]], rows.at[b], gsems[b])
            if j >= 1:
                pb = (j - 1) % NB
                gd[pb].wait()
                sd[pb] = pltpu.async_copy(
                    rows.at[pb], acc.at[didx.at[j - 1]], ssems[pb], add=True)
        lb = (SUP - 1) % NB
        gd[lb].wait()
        sd[lb] = pltpu.async_copy(
            rows.at[lb], acc.at[didx.at[SUP - 1]], ssems[lb], add=True)
        for b in range(NB):
            sd[b].wait()
        return carry

    lax.fori_loop(0, NSUPER, super_body, 0)
    plsc.subcore_barrier()

    pltpu.sync_copy(acc.at[pl.ds(stripe, ROWS_PER_TILE)],
                    out_hbm.at[cid, pl.ds(stripe, ROWS_PER_TILE)])


_sc_segsum = pl.kernel(
    _sc_segsum_body,
    out_type=jax.ShapeDtypeStruct((NC, N_PAD, D), jnp.float32),
    mesh=_SC_MESH,
    scratch_types=[
        pltpu.VMEM((SUP, CH), jnp.int32),
        pltpu.VMEM((SUP, CH), jnp.int32),
        pltpu.VMEM((NB, CH, D), jnp.float32),
        pltpu.VMEM_SHARED((N_PAD, D), jnp.float32),
        [pltpu.SemaphoreType.DMA] * NB,
        [pltpu.SemaphoreType.DMA] * NB,
    ],
    name="sc_segsum",
)


def _gin_mlp_body(hp, agg, eps, w1, b1, g1, be1, w2, b2, g2, be2, out):
    z = hp[...] * (1.0 + eps[0, 0]) + agg[0, :N, :] + agg[1, :N, :]
    for (w, b, g, be) in ((w1, b1, g1, be1), (w2, b2, g2, be2)):
        y = jnp.dot(z, w[...], preferred_element_type=jnp.float32) + b[...]
        m = jnp.mean(y, axis=0, keepdims=True)
        v = jnp.mean((y - m) ** 2, axis=0, keepdims=True)
        y = (y - m) * lax.rsqrt(v + 1e-5) * g[...] + be[...]
        z = jnp.where(y > 0, y, 0.01 * y)
    out[...] = jnp.where(z > 0, z, 0.01 * z)


_gin_mlp = pl.pallas_call(
    _gin_mlp_body,
    out_shape=jax.ShapeDtypeStruct((N, D), jnp.float32),
)


def _cls_body(h1, h2, h3, h4, bi, w640, b1r, wa, ba, wb, bbr, fw, fb, out):
    onehot = (lax.broadcasted_iota(jnp.int32, (N, G), 1) == bi[...])
    p = onehot.astype(jnp.float32)
    h4v = h4[...]
    pool = lax.dot_general(p, h4v, (((0,), (0,)), ((), ())),
                           preferred_element_type=jnp.float32)
    pooled = jnp.dot(p, pool, preferred_element_type=jnp.float32)
    w = w640[...]
    h = (jnp.dot(h1[...], w[0:128], preferred_element_type=jnp.float32)
         + jnp.dot(h2[...], w[128:256], preferred_element_type=jnp.float32)
         + jnp.dot(h3[...], w[256:384], preferred_element_type=jnp.float32)
         + jnp.dot(h4v, w[384:512], preferred_element_type=jnp.float32)
         + jnp.dot(pooled, w[512:640], preferred_element_type=jnp.float32)
         + b1r[...])
    for (wh, bh) in ((wa, ba), (wb, bbr)):
        h = jnp.dot(h, wh[...], preferred_element_type=jnp.float32) + bh[...]
        h = jnp.where(h > 0, h, 0.01 * h)
    o = jnp.sum(h * fw[...], axis=1, keepdims=True) + fb[0, 0]
    out[...] = 1.0 / (1.0 + jnp.exp(-o))


_classifier = pl.pallas_call(
    _cls_body,
    out_shape=jax.ShapeDtypeStruct((N, 1), jnp.float32),
)


def kernel(x, edge_index, batch, params):
    src = jnp.concatenate(
        [edge_index[0].astype(jnp.int32),
         jnp.arange(E_PAD - E, dtype=jnp.int32) % N]).reshape(NROWS2D, CH)
    dst = jnp.concatenate(
        [edge_index[1].astype(jnp.int32),
         N + jnp.arange(E_PAD - E, dtype=jnp.int32) % (N_PAD - N)]
    ).reshape(NROWS2D, CH)
    zeros = jnp.zeros((ROWS_PER_TILE, D), jnp.float32)
    bi = batch.astype(jnp.int32).reshape(N, 1)

    def row(v):
        return v.reshape(1, -1)

    h = x
    hs = []
    for i in range(4):
        layers = params['conv_mlps'][i]
        agg2 = _sc_segsum(h, src, dst, zeros)
        l1, l2 = layers[0], layers[1]
        h = _gin_mlp(h, agg2, params['eps'][i].reshape(1, 1),
                     l1['W'], row(l1['b']), row(l1['gamma']), row(l1['beta']),
                     l2['W'], row(l2['b']), row(l2['gamma']), row(l2['beta']))
        hs.append(h)

    out = _classifier(hs[0], hs[1], hs[2], hs[3], bi,
                      params['cls1_W'], row(params['cls1_b']),
                      params['cls_Ws'][0], row(params['cls_bs'][0]),
                      params['cls_Ws'][1], row(params['cls_bs'][1]),
                      row(params['fin_W'][:, 0]), params['fin_b'].reshape(1, 1))
    return out
